# Initial kernel scaffold; baseline (speedup 1.0000x reference)
import jax
import jax.numpy as jnp
from jax.experimental import pallas as pl


def kernel(x0, x1, keypointCoords0, keypointCoords1, W1, asrc1, adst1, b1, W2, asrc2, adst2, b2, W3, asrc3, adst3, b3, W4, asrc4, adst4, b4):
    raise NotImplementedError("write your pallas kernel here")


# speedup vs baseline: 4.5977x; 4.5977x over previous
import jax
import jax.numpy as jnp
from jax.experimental import pallas as pl

_SELF_BIG = 1e9
_SEL_BIG = 2e9
_NEG_BIG = -1e30
_K = 20


def _hst_body(x_ref, w_ref, asrc_ref, adst_ref, h_ref, s_ref, t_ref):
    x = x_ref[0]
    h = jnp.dot(x, w_ref[...], preferred_element_type=jnp.float32)
    h_ref[0] = h
    s_ref[0] = jnp.dot(h, asrc_ref[...], preferred_element_type=jnp.float32)
    t_ref[0] = jnp.dot(h, adst_ref[...], preferred_element_type=jnp.float32)


def _hst(x, w, asrc, adst):
    b, n, cin = x.shape
    c = w.shape[1]
    rh = 1024 if n % 1024 == 0 else n
    return pl.pallas_call(
        _hst_body,
        grid=(b, n // rh),
        in_specs=[
            pl.BlockSpec((1, rh, cin), lambda i, j: (i, j, 0)),
            pl.BlockSpec((cin, c), lambda i, j: (0, 0)),
            pl.BlockSpec((c, 1), lambda i, j: (0, 0)),
            pl.BlockSpec((c, 1), lambda i, j: (0, 0)),
        ],
        out_specs=[
            pl.BlockSpec((1, rh, c), lambda i, j: (i, j, 0)),
            pl.BlockSpec((1, rh, 1), lambda i, j: (i, j, 0)),
            pl.BlockSpec((1, rh, 1), lambda i, j: (i, j, 0)),
        ],
        out_shape=[
            jax.ShapeDtypeStruct((b, n, c), jnp.float32),
            jax.ShapeDtypeStruct((b, n, 1), jnp.float32),
            jax.ShapeDtypeStruct((b, n, 1), jnp.float32),
        ],
    )(x, w, asrc, adst)


def _gat_body(pt_ref, p_ref, h_ref, s_ref, t_ref, b_ref, f_ref, pp_ref, *, r, n, k):
    pt = pt_ref[0]
    q = p_ref[0]
    h = h_ref[0]
    s = s_ref[0]
    t = t_ref[0]
    pn = jnp.sum(pt * pt, axis=0, keepdims=True)
    qn = jnp.sum(q * q, axis=1, keepdims=True)
    dist = qn + pn - 2.0 * jnp.dot(q, pt, preferred_element_type=jnp.float32)
    cols = jax.lax.broadcasted_iota(jnp.int32, (r, n), 1)
    rows = pl.program_id(1) * r + jax.lax.broadcasted_iota(jnp.int32, (r, n), 0)
    dist = jnp.where(cols == rows, _SELF_BIG, dist)

    def body(_, d):
        m = jnp.min(d, axis=1, keepdims=True)
        return jnp.where(d == m, _SEL_BIG, d)

    dist = jax.lax.fori_loop(0, k, body, dist)
    sel = dist == _SEL_BIG

    lin = s + t
    lin = jnp.where(lin >= 0.0, lin, 0.2 * lin)
    e = jnp.where(sel, lin, _NEG_BIG)
    mx = jnp.max(e, axis=1, keepdims=True)
    z = jnp.exp(e - mx)
    alpha = z * (1.0 / jnp.sum(z, axis=1, keepdims=True))
    out = jnp.dot(alpha, h, preferred_element_type=jnp.float32)

    pc = jax.lax.broadcasted_iota(jnp.int32, (r // 4, r), 1)
    pr = jax.lax.broadcasted_iota(jnp.int32, (r // 4, r), 0)
    pm = jnp.where(pc // 4 == pr, 0.25, 0.0)
    f_ref[0] = jnp.dot(pm, out, preferred_element_type=jnp.float32) + b_ref[...]
    pp_ref[0] = jnp.dot(pm, q, preferred_element_type=jnp.float32)


def _pick_r(n):
    if n == 12288:
        return 64
    if n == 3072:
        return 256
    if n <= 1024:
        return n
    r = 512
    while n % r != 0:
        r //= 2
    return r


def _gat(pt, p, h, s, t, b):
    nb, n, c = h.shape
    r = _pick_r(n)
    import functools
    body = functools.partial(_gat_body, r=r, n=n, k=_K)
    return pl.pallas_call(
        body,
        grid=(nb, n // r),
        in_specs=[
            pl.BlockSpec((1, 3, n), lambda i, j: (i, 0, 0)),
            pl.BlockSpec((1, r, 3), lambda i, j: (i, j, 0)),
            pl.BlockSpec((1, n, c), lambda i, j: (i, 0, 0)),
            pl.BlockSpec((1, 1, n), lambda i, j: (i, 0, 0)),
            pl.BlockSpec((1, r, 1), lambda i, j: (i, j, 0)),
            pl.BlockSpec((1, c), lambda i, j: (0, 0)),
        ],
        out_specs=[
            pl.BlockSpec((1, r // 4, c), lambda i, j: (i, j, 0)),
            pl.BlockSpec((1, r // 4, 3), lambda i, j: (i, j, 0)),
        ],
        out_shape=[
            jax.ShapeDtypeStruct((nb, n // 4, c), jnp.float32),
            jax.ShapeDtypeStruct((nb, n // 4, 3), jnp.float32),
        ],
    )(pt, p, h, s, t, b)


def kernel(x0, x1, keypointCoords0, keypointCoords1,
           W1, asrc1, adst1, b1, W2, asrc2, adst2, b2,
           W3, asrc3, adst3, b3, W4, asrc4, adst4, b4):
    params = [(W1, asrc1, adst1, b1), (W2, asrc2, adst2, b2),
              (W3, asrc3, adst3, b3), (W4, asrc4, adst4, b4)]
    f = jnp.concatenate([x0, x1], axis=0)
    p = jnp.concatenate([keypointCoords0, keypointCoords1], axis=0)
    for w, asrc, adst, b in params:
        n = f.shape[1]
        h, s, t = _hst(f, w, asrc.reshape(-1, 1), adst.reshape(-1, 1))
        s = s.reshape(2, 1, n)
        pt = jnp.transpose(p, (0, 2, 1))
        f, p = _gat(pt, p, h, s, t, b.reshape(1, -1))
    return jnp.concatenate([f[0], f[1]], axis=0)


# speedup vs baseline: 11.0008x; 2.3927x over previous
import jax
import jax.numpy as jnp
from jax.experimental import pallas as pl

_SELF_BIG = 1e9
_SEL_BIG = 2e9
_NEG_BIG = -1e30
_K = 20


def _hst_body(x_ref, w_ref, asrc_ref, adst_ref, h_ref, s_ref, t_ref):
    x = x_ref[0]
    h = jnp.dot(x, w_ref[...], preferred_element_type=jnp.float32)
    h_ref[0] = h.astype(jnp.bfloat16)
    s_ref[0] = jnp.dot(h, asrc_ref[...], preferred_element_type=jnp.float32)
    t_ref[0] = jnp.dot(h, adst_ref[...], preferred_element_type=jnp.float32)


def _hst(x, w, asrc, adst):
    b, n, cin = x.shape
    c = w.shape[1]
    rh = 1024 if n % 1024 == 0 else n
    return pl.pallas_call(
        _hst_body,
        grid=(b, n // rh),
        in_specs=[
            pl.BlockSpec((1, rh, cin), lambda i, j: (i, j, 0)),
            pl.BlockSpec((cin, c), lambda i, j: (0, 0)),
            pl.BlockSpec((c, 1), lambda i, j: (0, 0)),
            pl.BlockSpec((c, 1), lambda i, j: (0, 0)),
        ],
        out_specs=[
            pl.BlockSpec((1, rh, c), lambda i, j: (i, j, 0)),
            pl.BlockSpec((1, rh, 1), lambda i, j: (i, j, 0)),
            pl.BlockSpec((1, rh, 1), lambda i, j: (i, j, 0)),
        ],
        out_shape=[
            jax.ShapeDtypeStruct((b, n, c), jnp.bfloat16),
            jax.ShapeDtypeStruct((b, n, 1), jnp.float32),
            jax.ShapeDtypeStruct((b, n, 1), jnp.float32),
        ],
    )(x, w, asrc, adst)


def _gat_body(pt_ref, p_ref, h_ref, s_ref, t_ref, b_ref, f_ref, pp_ref, *, r, n, k):
    pt = pt_ref[0]
    q = p_ref[0]
    h = h_ref[0]
    s = s_ref[0]
    t = t_ref[0]
    pn = jnp.sum(pt * pt, axis=0, keepdims=True)
    qn = jnp.sum(q * q, axis=1, keepdims=True)
    dist = qn + pn - 2.0 * jnp.dot(q, pt, preferred_element_type=jnp.float32)
    cols = jax.lax.broadcasted_iota(jnp.int32, (r, n), 1)
    rows = pl.program_id(1) * r + jax.lax.broadcasted_iota(jnp.int32, (r, n), 0)
    dist = jnp.where(cols == rows, _SELF_BIG, dist)

    for _ in range(k):
        m = jnp.min(dist, axis=1, keepdims=True)
        dist = jnp.where(dist == m, _SEL_BIG, dist)
    sel = dist == _SEL_BIG

    lin = s + t
    lin = jnp.where(lin >= 0.0, lin, 0.2 * lin)
    e = jnp.where(sel, lin, _NEG_BIG)
    mx = jnp.max(e, axis=1, keepdims=True)
    z = jnp.exp(e - mx)
    alpha = z * (1.0 / jnp.sum(z, axis=1, keepdims=True))
    out = jnp.dot(alpha.astype(jnp.bfloat16), h,
                  preferred_element_type=jnp.float32)

    pc = jax.lax.broadcasted_iota(jnp.int32, (r // 4, r), 1)
    pr = jax.lax.broadcasted_iota(jnp.int32, (r // 4, r), 0)
    pm = jnp.where(pc // 4 == pr, 0.25, 0.0)
    f_ref[0] = jnp.dot(pm, out, preferred_element_type=jnp.float32) + b_ref[...]
    pp_ref[0] = jnp.dot(pm, q, preferred_element_type=jnp.float32)


def _pick_r(n):
    if n == 12288:
        return 128
    if n == 3072:
        return 256
    if n <= 1024:
        return n
    r = 512
    while n % r != 0:
        r //= 2
    return r


def _gat(pt, p, h, s, t, b):
    nb, n, c = h.shape
    r = _pick_r(n)
    import functools
    body = functools.partial(_gat_body, r=r, n=n, k=_K)
    return pl.pallas_call(
        body,
        grid=(nb, n // r),
        in_specs=[
            pl.BlockSpec((1, 3, n), lambda i, j: (i, 0, 0)),
            pl.BlockSpec((1, r, 3), lambda i, j: (i, j, 0)),
            pl.BlockSpec((1, n, c), lambda i, j: (i, 0, 0)),
            pl.BlockSpec((1, 1, n), lambda i, j: (i, 0, 0)),
            pl.BlockSpec((1, r, 1), lambda i, j: (i, j, 0)),
            pl.BlockSpec((1, c), lambda i, j: (0, 0)),
        ],
        out_specs=[
            pl.BlockSpec((1, r // 4, c), lambda i, j: (i, j, 0)),
            pl.BlockSpec((1, r // 4, 3), lambda i, j: (i, j, 0)),
        ],
        out_shape=[
            jax.ShapeDtypeStruct((nb, n // 4, c), jnp.float32),
            jax.ShapeDtypeStruct((nb, n // 4, 3), jnp.float32),
        ],
    )(pt, p, h, s, t, b)


def kernel(x0, x1, keypointCoords0, keypointCoords1,
           W1, asrc1, adst1, b1, W2, asrc2, adst2, b2,
           W3, asrc3, adst3, b3, W4, asrc4, adst4, b4):
    params = [(W1, asrc1, adst1, b1), (W2, asrc2, adst2, b2),
              (W3, asrc3, adst3, b3), (W4, asrc4, adst4, b4)]
    f = jnp.concatenate([x0, x1], axis=0)
    p = jnp.concatenate([keypointCoords0, keypointCoords1], axis=0)
    for w, asrc, adst, b in params:
        n = f.shape[1]
        h, s, t = _hst(f, w, asrc.reshape(-1, 1), adst.reshape(-1, 1))
        s = s.reshape(2, 1, n)
        pt = jnp.transpose(p, (0, 2, 1))
        f, p = _gat(pt, p, h, s, t, b.reshape(1, -1))
    return jnp.concatenate([f[0], f[1]], axis=0)


# speedup vs baseline: 15.0888x; 1.3716x over previous
import jax
import jax.numpy as jnp
from jax.experimental import pallas as pl

_SELF_BIG = 1e9
_SEL_BIG = 2e9
_NEG_BIG = -1e30
_K = 20


def _hst_body(x_ref, w_ref, asrc_ref, adst_ref, h_ref, s_ref, t_ref):
    x = x_ref[0]
    h = jnp.dot(x, w_ref[...], preferred_element_type=jnp.float32)
    h_ref[0] = h.astype(jnp.bfloat16)
    s_ref[0] = jnp.dot(h, asrc_ref[...], preferred_element_type=jnp.float32)
    t_ref[0] = jnp.dot(h, adst_ref[...], preferred_element_type=jnp.float32)


def _hst(x, w, asrc, adst):
    b, n, cin = x.shape
    c = w.shape[1]
    rh = 1024 if n % 1024 == 0 else n
    return pl.pallas_call(
        _hst_body,
        grid=(b, n // rh),
        in_specs=[
            pl.BlockSpec((1, rh, cin), lambda i, j: (i, j, 0)),
            pl.BlockSpec((cin, c), lambda i, j: (0, 0)),
            pl.BlockSpec((c, 1), lambda i, j: (0, 0)),
            pl.BlockSpec((c, 1), lambda i, j: (0, 0)),
        ],
        out_specs=[
            pl.BlockSpec((1, rh, c), lambda i, j: (i, j, 0)),
            pl.BlockSpec((1, rh, 1), lambda i, j: (i, j, 0)),
            pl.BlockSpec((1, rh, 1), lambda i, j: (i, j, 0)),
        ],
        out_shape=[
            jax.ShapeDtypeStruct((b, n, c), jnp.bfloat16),
            jax.ShapeDtypeStruct((b, n, 1), jnp.float32),
            jax.ShapeDtypeStruct((b, n, 1), jnp.float32),
        ],
    )(x, w, asrc, adst)


def _gat_body(bigp_ref, p_ref, qt_ref, h_ref, s_ref, t_ref, b_ref,
              f_ref, pp_ref, *, r, n, k):
    bigp = bigp_ref[0]
    q = p_ref[0]
    qt = qt_ref[0]
    h = h_ref[0]
    st = s_ref[0]
    tt = t_ref[0]
    pn = jnp.sum(bigp * bigp, axis=1, keepdims=True)
    qn = jnp.sum(qt * qt, axis=0, keepdims=True)
    dist = pn + qn - 2.0 * jnp.dot(bigp, qt, preferred_element_type=jnp.float32)
    rows = jax.lax.broadcasted_iota(jnp.int32, (n, r), 0)
    cols = pl.program_id(1) * r + jax.lax.broadcasted_iota(jnp.int32, (n, r), 1)
    dist = jnp.where(rows == cols, _SELF_BIG, dist)

    if n >= 3072 and n % 96 == 0:
        nc = 96
        cs = n // nc
        d3 = dist.reshape(nc, cs, r)
        gs = []
        dcur = d3
        for i in range(4):
            g = jnp.min(dcur, axis=1)
            gs.append(g)
            if i < 3:
                dcur = jnp.where(dcur == g[:, None, :], _SEL_BIG, dcur)
        cnt = jnp.zeros((nc, r), jnp.int32)
        cur = gs[0]
        tau = None
        for _ in range(k):
            tau = jnp.min(cur, axis=0, keepdims=True)
            pick = cur == tau
            cnt = cnt + pick.astype(jnp.int32)
            nxt = jnp.where(cnt == 1, gs[1],
                  jnp.where(cnt == 2, gs[2],
                  jnp.where(cnt == 3, gs[3], _SEL_BIG)))
            cur = jnp.where(pick, nxt, cur)
        sel = dist <= tau
    else:
        for _ in range(k):
            m = jnp.min(dist, axis=0, keepdims=True)
            dist = jnp.where(dist == m, _SEL_BIG, dist)
        sel = dist == _SEL_BIG

    lin = st + tt
    lin = jnp.where(lin >= 0.0, lin, 0.2 * lin)
    z = jnp.where(sel, jnp.exp(lin), 0.0)
    alpha = z * (1.0 / jnp.sum(z, axis=0, keepdims=True))
    out = jax.lax.dot_general(alpha.astype(jnp.bfloat16), h,
                              (((0,), (0,)), ((), ())),
                              preferred_element_type=jnp.float32)

    pc = jax.lax.broadcasted_iota(jnp.int32, (r // 4, r), 1)
    pr = jax.lax.broadcasted_iota(jnp.int32, (r // 4, r), 0)
    pm = jnp.where(pc // 4 == pr, 0.25, 0.0)
    f_ref[0] = jnp.dot(pm, out, preferred_element_type=jnp.float32) + b_ref[...]
    pp_ref[0] = jnp.dot(pm, q, preferred_element_type=jnp.float32)


def _pick_r(n):
    if n == 12288:
        return 128
    if n == 3072:
        return 256
    if n <= 1024:
        return n
    r = 512
    while n % r != 0:
        r //= 2
    return r


def _gat(p, pt, h, s, t, b):
    nb, n, c = h.shape
    r = _pick_r(n)
    import functools
    body = functools.partial(_gat_body, r=r, n=n, k=_K)
    return pl.pallas_call(
        body,
        grid=(nb, n // r),
        in_specs=[
            pl.BlockSpec((1, n, 3), lambda i, j: (i, 0, 0)),
            pl.BlockSpec((1, r, 3), lambda i, j: (i, j, 0)),
            pl.BlockSpec((1, 3, r), lambda i, j: (i, 0, j)),
            pl.BlockSpec((1, n, c), lambda i, j: (i, 0, 0)),
            pl.BlockSpec((1, n, 1), lambda i, j: (i, 0, 0)),
            pl.BlockSpec((1, 1, r), lambda i, j: (i, 0, j)),
            pl.BlockSpec((1, c), lambda i, j: (0, 0)),
        ],
        out_specs=[
            pl.BlockSpec((1, r // 4, c), lambda i, j: (i, j, 0)),
            pl.BlockSpec((1, r // 4, 3), lambda i, j: (i, j, 0)),
        ],
        out_shape=[
            jax.ShapeDtypeStruct((nb, n // 4, c), jnp.float32),
            jax.ShapeDtypeStruct((nb, n // 4, 3), jnp.float32),
        ],
    )(p, p, pt, h, s, t, b)


def kernel(x0, x1, keypointCoords0, keypointCoords1,
           W1, asrc1, adst1, b1, W2, asrc2, adst2, b2,
           W3, asrc3, adst3, b3, W4, asrc4, adst4, b4):
    params = [(W1, asrc1, adst1, b1), (W2, asrc2, adst2, b2),
              (W3, asrc3, adst3, b3), (W4, asrc4, adst4, b4)]
    f = jnp.concatenate([x0, x1], axis=0)
    p = jnp.concatenate([keypointCoords0, keypointCoords1], axis=0)
    for w, asrc, adst, b in params:
        n = f.shape[1]
        h, s, t = _hst(f, w, asrc.reshape(-1, 1), adst.reshape(-1, 1))
        t = t.reshape(2, 1, n)
        pt = jnp.transpose(p, (0, 2, 1))
        f, p = _gat(p, pt, h, s, t, b.reshape(1, -1))
    return jnp.concatenate([f[0], f[1]], axis=0)


# speedup vs baseline: 18.4359x; 1.2218x over previous
import jax
import jax.numpy as jnp
from jax.experimental import pallas as pl

_SELF_BIG = 1e9
_SEL_BIG = 2e9
_NEG_BIG = -1e30
_K = 20


def _hst_body(x_ref, w_ref, asrc_ref, adst_ref, h_ref, s_ref, t_ref):
    x = x_ref[0]
    h = jnp.dot(x, w_ref[...], preferred_element_type=jnp.float32)
    h_ref[0] = h.astype(jnp.bfloat16)
    s_ref[0] = jnp.dot(h, asrc_ref[...], preferred_element_type=jnp.float32)
    t_ref[0] = jnp.dot(h, adst_ref[...], preferred_element_type=jnp.float32)


def _hst(x, w, asrc, adst):
    b, n, cin = x.shape
    c = w.shape[1]
    rh = 1024 if n % 1024 == 0 else n
    return pl.pallas_call(
        _hst_body,
        grid=(b, n // rh),
        in_specs=[
            pl.BlockSpec((1, rh, cin), lambda i, j: (i, j, 0)),
            pl.BlockSpec((cin, c), lambda i, j: (0, 0)),
            pl.BlockSpec((c, 1), lambda i, j: (0, 0)),
            pl.BlockSpec((c, 1), lambda i, j: (0, 0)),
        ],
        out_specs=[
            pl.BlockSpec((1, rh, c), lambda i, j: (i, j, 0)),
            pl.BlockSpec((1, rh, 1), lambda i, j: (i, j, 0)),
            pl.BlockSpec((1, rh, 1), lambda i, j: (i, j, 0)),
        ],
        out_shape=[
            jax.ShapeDtypeStruct((b, n, c), jnp.bfloat16),
            jax.ShapeDtypeStruct((b, n, 1), jnp.float32),
            jax.ShapeDtypeStruct((b, n, 1), jnp.float32),
        ],
    )(x, w, asrc, adst)


def _gat_body(bigp_ref, p_ref, qt_ref, h_ref, s_ref, t_ref, b_ref,
              f_ref, pp_ref, *, r, n, k):
    bigp = bigp_ref[0]
    q = p_ref[0]
    qt = qt_ref[0]
    h = h_ref[0]
    st = s_ref[0]
    tt = t_ref[0]
    pn = jnp.sum(bigp * bigp, axis=1, keepdims=True)
    qn = jnp.sum(qt * qt, axis=0, keepdims=True)
    dist = pn + qn - 2.0 * jnp.dot(bigp, qt, preferred_element_type=jnp.float32)
    rows = jax.lax.broadcasted_iota(jnp.int32, (n, r), 0)
    cols = pl.program_id(1) * r + jax.lax.broadcasted_iota(jnp.int32, (n, r), 1)
    dist = jnp.where(rows == cols, _SELF_BIG, dist)

    if n >= 3072 and n % 96 == 0:
        nc = 96
        cs = n // nc
        d3 = dist.reshape(nc, cs, r)
        gs = []
        dcur = d3
        for i in range(4):
            g = jnp.min(dcur, axis=1)
            gs.append(g)
            if i < 3:
                dcur = jnp.where(dcur == g[:, None, :], _SEL_BIG, dcur)
        cnt = jnp.zeros((nc, r), jnp.int32)
        cur = gs[0]
        tau = None
        for _ in range(k):
            tau = jnp.min(cur, axis=0, keepdims=True)
            pick = cur == tau
            cnt = cnt + pick.astype(jnp.int32)
            nxt = jnp.where(cnt == 1, gs[1],
                  jnp.where(cnt == 2, gs[2],
                  jnp.where(cnt == 3, gs[3], _SEL_BIG)))
            cur = jnp.where(pick, nxt, cur)
        sel = dist <= tau
    else:
        for _ in range(k):
            m = jnp.min(dist, axis=0, keepdims=True)
            dist = jnp.where(dist == m, _SEL_BIG, dist)
        sel = dist == _SEL_BIG

    lin = st + tt
    lin = jnp.where(lin >= 0.0, lin, 0.2 * lin)
    z = jnp.where(sel, jnp.exp(lin), 0.0).astype(jnp.bfloat16)
    denom = jnp.sum(z.astype(jnp.float32), axis=0, keepdims=True)
    out = jax.lax.dot_general(z, h, (((0,), (0,)), ((), ())),
                              preferred_element_type=jnp.float32)
    out = out * (1.0 / denom).reshape(r, 1)

    pc = jax.lax.broadcasted_iota(jnp.int32, (r // 4, r), 1)
    pr = jax.lax.broadcasted_iota(jnp.int32, (r // 4, r), 0)
    pm = jnp.where(pc // 4 == pr, 0.25, 0.0)
    f_ref[0] = jnp.dot(pm, out, preferred_element_type=jnp.float32) + b_ref[...]
    pp_ref[0] = jnp.dot(pm, q, preferred_element_type=jnp.float32)


def _pick_r(n):
    if n == 12288:
        return 256
    if n == 3072:
        return 256
    if n <= 1024:
        return n
    r = 512
    while n % r != 0:
        r //= 2
    return r


def _gat(p, pt, h, s, t, b):
    nb, n, c = h.shape
    r = _pick_r(n)
    import functools
    body = functools.partial(_gat_body, r=r, n=n, k=_K)
    return pl.pallas_call(
        body,
        grid=(nb, n // r),
        in_specs=[
            pl.BlockSpec((1, n, 3), lambda i, j: (i, 0, 0)),
            pl.BlockSpec((1, r, 3), lambda i, j: (i, j, 0)),
            pl.BlockSpec((1, 3, r), lambda i, j: (i, 0, j)),
            pl.BlockSpec((1, n, c), lambda i, j: (i, 0, 0)),
            pl.BlockSpec((1, n, 1), lambda i, j: (i, 0, 0)),
            pl.BlockSpec((1, 1, r), lambda i, j: (i, 0, j)),
            pl.BlockSpec((1, c), lambda i, j: (0, 0)),
        ],
        out_specs=[
            pl.BlockSpec((1, r // 4, c), lambda i, j: (i, j, 0)),
            pl.BlockSpec((1, r // 4, 3), lambda i, j: (i, j, 0)),
        ],
        out_shape=[
            jax.ShapeDtypeStruct((nb, n // 4, c), jnp.float32),
            jax.ShapeDtypeStruct((nb, n // 4, 3), jnp.float32),
        ],
    )(p, p, pt, h, s, t, b)


def kernel(x0, x1, keypointCoords0, keypointCoords1,
           W1, asrc1, adst1, b1, W2, asrc2, adst2, b2,
           W3, asrc3, adst3, b3, W4, asrc4, adst4, b4):
    params = [(W1, asrc1, adst1, b1), (W2, asrc2, adst2, b2),
              (W3, asrc3, adst3, b3), (W4, asrc4, adst4, b4)]
    f = jnp.concatenate([x0, x1], axis=0)
    p = jnp.concatenate([keypointCoords0, keypointCoords1], axis=0)
    for w, asrc, adst, b in params:
        n = f.shape[1]
        h, s, t = _hst(f, w, asrc.reshape(-1, 1), adst.reshape(-1, 1))
        t = t.reshape(2, 1, n)
        pt = jnp.transpose(p, (0, 2, 1))
        f, p = _gat(p, pt, h, s, t, b.reshape(1, -1))
    return jnp.concatenate([f[0], f[1]], axis=0)


# speedup vs baseline: 20.4295x; 1.1081x over previous
import jax
import jax.numpy as jnp
from jax.experimental import pallas as pl

_SELF_BIG = 1e9
_SEL_BIG = 2e9
_NEG_BIG = -1e30
_K = 20


def _hst_body(x_ref, p_ref, w_ref, asrc_ref, adst_ref, h_ref, s_ref, t_ref,
              pn_ref):
    x = x_ref[0]
    h = jnp.dot(x, w_ref[...], preferred_element_type=jnp.float32)
    h_ref[0] = h.astype(jnp.bfloat16)
    s_ref[0] = jnp.dot(h, asrc_ref[...], preferred_element_type=jnp.float32)
    t_ref[0] = jnp.dot(h, adst_ref[...], preferred_element_type=jnp.float32)
    p = p_ref[0]
    pn_ref[0] = jnp.sum(p * p, axis=1, keepdims=True)


def _hst(x, p, w, asrc, adst):
    b, n, cin = x.shape
    c = w.shape[1]
    rh = 1024 if n % 1024 == 0 else n
    return pl.pallas_call(
        _hst_body,
        grid=(b, n // rh),
        in_specs=[
            pl.BlockSpec((1, rh, cin), lambda i, j: (i, j, 0)),
            pl.BlockSpec((1, rh, 3), lambda i, j: (i, j, 0)),
            pl.BlockSpec((cin, c), lambda i, j: (0, 0)),
            pl.BlockSpec((c, 1), lambda i, j: (0, 0)),
            pl.BlockSpec((c, 1), lambda i, j: (0, 0)),
        ],
        out_specs=[
            pl.BlockSpec((1, rh, c), lambda i, j: (i, j, 0)),
            pl.BlockSpec((1, rh, 1), lambda i, j: (i, j, 0)),
            pl.BlockSpec((1, rh, 1), lambda i, j: (i, j, 0)),
            pl.BlockSpec((1, rh, 1), lambda i, j: (i, j, 0)),
        ],
        out_shape=[
            jax.ShapeDtypeStruct((b, n, c), jnp.bfloat16),
            jax.ShapeDtypeStruct((b, n, 1), jnp.float32),
            jax.ShapeDtypeStruct((b, n, 1), jnp.float32),
            jax.ShapeDtypeStruct((b, n, 1), jnp.float32),
        ],
    )(x, p, w, asrc, adst)


def _gat_body(paug_ref, qaug_ref, p_ref, h_ref, s_ref, t_ref, b_ref,
              f_ref, pp_ref, *, r, n, k):
    q = p_ref[0]
    h = h_ref[0]
    st = s_ref[0]
    tt = t_ref[0]
    dist = jnp.dot(paug_ref[0], qaug_ref[0], preferred_element_type=jnp.float32)
    rows = jax.lax.broadcasted_iota(jnp.int32, (n, r), 0)
    cols = pl.program_id(1) * r + jax.lax.broadcasted_iota(jnp.int32, (n, r), 1)
    dist = jnp.where(rows == cols, _SELF_BIG, dist)

    if n >= 3072 and n % 96 == 0:
        nc = 96
        cs = n // nc
        d3 = dist.reshape(nc, cs, r)
        gs = []
        dcur = d3
        for i in range(4):
            g = jnp.min(dcur, axis=1)
            gs.append(g)
            if i < 3:
                dcur = jnp.where(dcur == g[:, None, :], _SEL_BIG, dcur)
        cur, q1, q2, q3 = gs
        tau = None
        for _ in range(k):
            tau = jnp.min(cur, axis=0, keepdims=True)
            pick = cur == tau
            cur = jnp.where(pick, q1, cur)
            q1 = jnp.where(pick, q2, q1)
            q2 = jnp.where(pick, q3, q2)
            q3 = jnp.where(pick, _SEL_BIG, q3)
        sel = dist <= tau
    else:
        for _ in range(k):
            m = jnp.min(dist, axis=0, keepdims=True)
            dist = jnp.where(dist == m, _SEL_BIG, dist)
        sel = dist == _SEL_BIG

    lin = st + tt
    lin = jnp.where(lin >= 0.0, lin, 0.2 * lin)
    z = jnp.where(sel, jnp.exp(lin), 0.0).astype(jnp.bfloat16)
    denom = jnp.sum(z.astype(jnp.float32), axis=0, keepdims=True)
    out = jax.lax.dot_general(z, h, (((0,), (0,)), ((), ())),
                              preferred_element_type=jnp.float32)
    out = out * (1.0 / denom).reshape(r, 1)

    pc = jax.lax.broadcasted_iota(jnp.int32, (r // 4, r), 1)
    pr = jax.lax.broadcasted_iota(jnp.int32, (r // 4, r), 0)
    pm = jnp.where(pc // 4 == pr, 0.25, 0.0)
    f_ref[0] = jnp.dot(pm, out, preferred_element_type=jnp.float32) + b_ref[...]
    pp_ref[0] = jnp.dot(pm, q, preferred_element_type=jnp.float32)


def _pick_r(n):
    if n == 12288:
        return 256
    if n == 3072:
        return 256
    if n <= 1024:
        return n
    r = 512
    while n % r != 0:
        r //= 2
    return r


def _gat(paug, qaug, p, h, s, t, b):
    nb, n, c = h.shape
    r = _pick_r(n)
    import functools
    body = functools.partial(_gat_body, r=r, n=n, k=_K)
    return pl.pallas_call(
        body,
        grid=(nb, n // r),
        in_specs=[
            pl.BlockSpec((1, n, 5), lambda i, j: (i, 0, 0)),
            pl.BlockSpec((1, 5, r), lambda i, j: (i, 0, j)),
            pl.BlockSpec((1, r, 3), lambda i, j: (i, j, 0)),
            pl.BlockSpec((1, n, c), lambda i, j: (i, 0, 0)),
            pl.BlockSpec((1, n, 1), lambda i, j: (i, 0, 0)),
            pl.BlockSpec((1, 1, r), lambda i, j: (i, 0, j)),
            pl.BlockSpec((1, c), lambda i, j: (0, 0)),
        ],
        out_specs=[
            pl.BlockSpec((1, r // 4, c), lambda i, j: (i, j, 0)),
            pl.BlockSpec((1, r // 4, 3), lambda i, j: (i, j, 0)),
        ],
        out_shape=[
            jax.ShapeDtypeStruct((nb, n // 4, c), jnp.float32),
            jax.ShapeDtypeStruct((nb, n // 4, 3), jnp.float32),
        ],
    )(paug, qaug, p, h, s, t, b)


def kernel(x0, x1, keypointCoords0, keypointCoords1,
           W1, asrc1, adst1, b1, W2, asrc2, adst2, b2,
           W3, asrc3, adst3, b3, W4, asrc4, adst4, b4):
    params = [(W1, asrc1, adst1, b1), (W2, asrc2, adst2, b2),
              (W3, asrc3, adst3, b3), (W4, asrc4, adst4, b4)]
    f = jnp.concatenate([x0, x1], axis=0)
    p = jnp.concatenate([keypointCoords0, keypointCoords1], axis=0)
    for w, asrc, adst, b in params:
        n = f.shape[1]
        h, s, t, pn = _hst(f, p, w, asrc.reshape(-1, 1), adst.reshape(-1, 1))
        t = t.reshape(2, 1, n)
        one = jnp.ones_like(pn)
        paug = jnp.concatenate([-2.0 * p, pn, one], axis=-1)
        qaug = jnp.transpose(jnp.concatenate([p, one, pn], axis=-1), (0, 2, 1))
        f, p = _gat(paug, qaug, p, h, s, t, b.reshape(1, -1))
    return jnp.concatenate([f[0], f[1]], axis=0)
